# Initial kernel scaffold; baseline (speedup 1.0000x reference)
import jax
import jax.numpy as jnp
from jax.experimental import pallas as pl


def kernel(ids, features, adj, W_self0, W_neigh0, b0, W_self1, W_neigh1, b1, fc_W, fc_b):
    raise NotImplementedError("write your pallas kernel here")


# speedup vs baseline: 3.6599x; 3.6599x over previous
import functools

import jax
import jax.numpy as jnp
from jax import lax
from jax.experimental import pallas as pl
from jax.experimental.pallas import tpu as pltpu, tpu_sc as plsc

N_NODE = 100000
IN_DIM = 128
HID = 128
N_CLASS = 41
BATCH = 1024
MAX_DEG = 32
NS1 = 25
NS2 = 10

NW = 32
B_PER_W = BATCH // NW
P_PER_W = B_PER_W * NS1
CH = 80
ND = CH // NS2
N1 = P_PER_W // CH
N2 = P_PER_W * NS2 // CH
LANES = 16
APR = 128 // MAX_DEG


def _sc_body(ids_hbm, feat_hbm, adj_hbm, x0_hbm, x1_hbm, m2_hbm,
             ids_v, adj1_v, cur1_v, x0_v, aidx1_v, aidx_v, rbuf_v, cur2_v,
             stage_v, m2b_v, sem0, sem1):
    wid = lax.axis_index("s") * 2 + lax.axis_index("c")
    bbase = wid * B_PER_W
    pbase = wid * P_PER_W

    pltpu.sync_copy(ids_hbm.at[pl.ds(pl.multiple_of(bbase, 8), B_PER_W)], ids_v)

    pltpu.async_copy(feat_hbm.at[ids_v], x0_v, sem1).wait()
    pltpu.sync_copy(x0_v, x0_hbm.at[pl.ds(pl.multiple_of(bbase, 8), B_PER_W)])

    for k in range(B_PER_W // LANES):
        aidx1_v[pl.ds(k * LANES, LANES)] = \
            ids_v[pl.ds(k * LANES, LANES)] >> 2
    pltpu.async_copy(adj_hbm.at[aidx1_v], adj1_v, sem0).wait()

    def build1(k, carry):
        p = lax.iota(jnp.int32, LANES) + k * LANES
        r = (p * 5243) >> 17
        c = ((plsc.load_gather(ids_v, [r]) & (APR - 1)) * MAX_DEG
             + p - r * NS1)
        cur1_v[pl.ds(k * LANES, LANES)] = plsc.load_gather(adj1_v, [r, c])
        return carry
    lax.fori_loop(0, P_PER_W // LANES, build1, 0)

    def x1_loop(g, carry):
        pltpu.async_copy(feat_hbm.at[cur1_v.at[pl.ds(g * CH, CH)]],
                         stage_v.at[0], sem0).wait()
        pltpu.sync_copy(stage_v.at[0],
                        x1_hbm.at[pl.ds(pl.multiple_of(pbase + g * CH, 8), CH)])
        return carry
    lax.fori_loop(0, N1, x1_loop, 0)

    def c2_loop(g, carry):
        def cidx(k, c2):
            aidx_v[pl.ds(k * LANES, LANES)] = \
                cur1_v[pl.ds(g * CH + k * LANES, LANES)] >> 2
            return c2
        lax.fori_loop(0, CH // LANES, cidx, 0)
        pltpu.async_copy(adj_hbm.at[aidx_v], rbuf_v, sem0).wait()

        def ext(k, c2):
            q = lax.iota(jnp.int32, LANES) + k * LANES
            r = (q * 6554) >> 16
            c = ((plsc.load_gather(cur1_v, [g * CH + r]) & (APR - 1)) * MAX_DEG
                 + q - r * NS2)
            cur2_v[pl.ds(g * CH * NS2 + k * LANES, LANES)] = \
                plsc.load_gather(rbuf_v, [r, c])
            return c2
        lax.fori_loop(0, CH * NS2 // LANES, ext, 0)
        return carry
    lax.fori_loop(0, N1, c2_loop, 0)

    def m2_outer(t, carry):
        def m2_loop(u, c1):
            g = t * N1 + u
            pltpu.async_copy(feat_hbm.at[cur2_v.at[pl.ds(g * CH, CH)]],
                             stage_v.at[0], sem0).wait()
            dbase = u * ND

            def red(d, c2):
                for c in range(IN_DIM // LANES):
                    acc = stage_v[0, d * NS2, pl.ds(c * LANES, LANES)]
                    for j in range(1, NS2):
                        acc = acc + stage_v[0, d * NS2 + j,
                                            pl.ds(c * LANES, LANES)]
                    m2b_v[dbase + d, pl.ds(c * LANES, LANES)] = acc * (1.0 / NS2)
                return c2
            lax.fori_loop(0, ND, red, 0)
            return c1
        lax.fori_loop(0, N1, m2_loop, 0)
        pltpu.sync_copy(
            m2b_v, m2_hbm.at[pl.ds(pl.multiple_of(pbase + t * CH, 8), CH)])
        return carry
    lax.fori_loop(0, N1, m2_outer, 0)


@functools.partial(
    pl.kernel,
    out_type=[
        jax.ShapeDtypeStruct((BATCH, IN_DIM), jnp.float32),
        jax.ShapeDtypeStruct((BATCH * NS1, IN_DIM), jnp.float32),
        jax.ShapeDtypeStruct((BATCH * NS1, IN_DIM), jnp.float32),
    ],
    mesh=plsc.VectorSubcoreMesh(core_axis_name="c", subcore_axis_name="s"),
    scratch_types=[
        pltpu.VMEM((B_PER_W,), jnp.int32),
        pltpu.VMEM((B_PER_W, APR * MAX_DEG), jnp.int32),
        pltpu.VMEM((P_PER_W,), jnp.int32),
        pltpu.VMEM((B_PER_W, IN_DIM), jnp.float32),
        pltpu.VMEM((B_PER_W,), jnp.int32),
        pltpu.VMEM((CH,), jnp.int32),
        pltpu.VMEM((CH, APR * MAX_DEG), jnp.int32),
        pltpu.VMEM((P_PER_W * NS2,), jnp.int32),
        pltpu.VMEM((2, CH, IN_DIM), jnp.float32),
        pltpu.VMEM((CH, IN_DIM), jnp.float32),
        pltpu.SemaphoreType.DMA,
        pltpu.SemaphoreType.DMA,
    ],
    compiler_params=pltpu.CompilerParams(needs_layout_passes=False),
)
def _sc_gather(*refs):
    _sc_body(*refs)


BB = 128
GRID = BATCH // BB


def _tc_body(x0_ref, x1_ref, m2_ref, ws0_ref, wn0_ref, b0_ref,
             ws1_ref, wn1_ref, b1_ref, fcw_ref, fcb_ref, out_ref):
    hp = jax.lax.Precision.HIGHEST
    x1 = x1_ref[...]
    h1 = jnp.maximum(
        jnp.dot(x1, ws0_ref[...], precision=hp)
        + jnp.dot(m2_ref[...], wn0_ref[...], precision=hp) + b0_ref[...], 0.0)
    m1 = jnp.mean(x1.reshape(BB, NS1, IN_DIM), axis=1)
    mh1 = jnp.mean(h1.reshape(BB, NS1, HID), axis=1)
    h0 = jnp.maximum(
        jnp.dot(x0_ref[...], ws0_ref[...], precision=hp)
        + jnp.dot(m1, wn0_ref[...], precision=hp) + b0_ref[...], 0.0)
    z = jnp.maximum(
        jnp.dot(h0, ws1_ref[...], precision=hp)
        + jnp.dot(mh1, wn1_ref[...], precision=hp) + b1_ref[...], 0.0)
    out_ref[...] = jnp.dot(z, fcw_ref[...], precision=hp) + fcb_ref[...]


def _tc_dense(x0, x1, m2, ws0, wn0, b0, ws1, wn1, b1, fcw, fcb):
    full = lambda shape: pl.BlockSpec(shape, lambda i: (0, 0))
    return pl.pallas_call(
        _tc_body,
        grid=(GRID,),
        in_specs=[
            pl.BlockSpec((BB, IN_DIM), lambda i: (i, 0)),
            pl.BlockSpec((BB * NS1, IN_DIM), lambda i: (i, 0)),
            pl.BlockSpec((BB * NS1, IN_DIM), lambda i: (i, 0)),
            full((IN_DIM, HID)),
            full((IN_DIM, HID)),
            full((1, HID)),
            full((HID, HID)),
            full((HID, HID)),
            full((1, HID)),
            full((HID, N_CLASS)),
            full((1, N_CLASS)),
        ],
        out_specs=pl.BlockSpec((BB, N_CLASS), lambda i: (i, 0)),
        out_shape=jax.ShapeDtypeStruct((BATCH, N_CLASS), jnp.float32),
    )(x0, x1, m2, ws0, wn0, b0, ws1, wn1, b1, fcw, fcb)


def kernel(ids, features, adj, W_self0, W_neigh0, b0, W_self1, W_neigh1, b1,
           fc_W, fc_b):
    ids = ids.astype(jnp.int32)
    adj_r = adj.astype(jnp.int32).reshape(N_NODE // APR, APR * MAX_DEG)
    x0, x1, m2 = _sc_gather(ids, features, adj_r)
    return _tc_dense(x0, x1, m2, W_self0, W_neigh0, b0.reshape(1, HID),
                     W_self1, W_neigh1, b1.reshape(1, HID),
                     fc_W, fc_b.reshape(1, N_CLASS))


# speedup vs baseline: 4.8202x; 1.3170x over previous
import functools

import jax
import jax.numpy as jnp
from jax import lax
from jax.experimental import pallas as pl
from jax.experimental.pallas import tpu as pltpu, tpu_sc as plsc

N_NODE = 100000
IN_DIM = 128
HID = 128
N_CLASS = 41
BATCH = 1024
MAX_DEG = 32
NS1 = 25
NS2 = 10

NW = 32
NSUB = 16
B_PER_W = BATCH // NW
P_PER_W = B_PER_W * NS1
CH = 80
ND = CH // NS2
N1 = P_PER_W // CH
N2 = P_PER_W * NS2 // CH
LANES = 16
APR = 128 // MAX_DEG
NPASS = 2
ROWS_P = P_PER_W // NPASS
CHUNKS_P = N2 // NPASS


def _sc_body(ids_hbm, feat_hbm, adj_hbm, x0_hbm, x1_hbm, m2_hbm,
             ids_v, adj1_v, cur1_v, x0_v, aidx1_v, aidx_v, rbuf_v, cur2_v,
             stage_v, didx_v, pat_v, acc_sh, sem0, sem1,
             gsem0, gsem1, ssem0, ssem1):
    cid = lax.axis_index("c")
    sid = lax.axis_index("s")
    wid = sid * 2 + cid
    bbase = wid * B_PER_W
    pbase = wid * P_PER_W

    pltpu.sync_copy(ids_hbm.at[pl.ds(pl.multiple_of(bbase, 8), B_PER_W)], ids_v)

    pltpu.async_copy(feat_hbm.at[ids_v], x0_v, sem1).wait()
    pltpu.sync_copy(x0_v, x0_hbm.at[pl.ds(pl.multiple_of(bbase, 8), B_PER_W)])

    for k in range(B_PER_W // LANES):
        aidx1_v[pl.ds(k * LANES, LANES)] = \
            ids_v[pl.ds(k * LANES, LANES)] >> 2
    pltpu.async_copy(adj_hbm.at[aidx1_v], adj1_v, sem0).wait()

    def build1(k, carry):
        p = lax.iota(jnp.int32, LANES) + k * LANES
        r = (p * 5243) >> 17
        c = ((plsc.load_gather(ids_v, [r]) & (APR - 1)) * MAX_DEG
             + p - r * NS1)
        cur1_v[pl.ds(k * LANES, LANES)] = plsc.load_gather(adj1_v, [r, c])
        return carry
    lax.fori_loop(0, P_PER_W // LANES, build1, 0)

    def x1_loop(g, carry):
        pltpu.async_copy(feat_hbm.at[cur1_v.at[pl.ds(g * CH, CH)]],
                         stage_v.at[0], sem0).wait()
        pltpu.sync_copy(stage_v.at[0],
                        x1_hbm.at[pl.ds(pl.multiple_of(pbase + g * CH, 8), CH)])
        return carry
    lax.fori_loop(0, N1, x1_loop, 0)

    def c2_loop(g, carry):
        def cidx(k, c2):
            aidx_v[pl.ds(k * LANES, LANES)] = \
                cur1_v[pl.ds(g * CH + k * LANES, LANES)] >> 2
            return c2
        lax.fori_loop(0, CH // LANES, cidx, 0)
        pltpu.async_copy(adj_hbm.at[aidx_v], rbuf_v, sem0).wait()

        def ext(k, c2):
            q = lax.iota(jnp.int32, LANES) + k * LANES
            r = (q * 6554) >> 16
            c = ((plsc.load_gather(cur1_v, [g * CH + r]) & (APR - 1)) * MAX_DEG
                 + q - r * NS2)
            cur2_v[pl.ds(g * CH * NS2 + k * LANES, LANES)] = \
                plsc.load_gather(rbuf_v, [r, c])
            return c2
        lax.fori_loop(0, CH * NS2 // LANES, ext, 0)
        return carry
    lax.fori_loop(0, N1, c2_loop, 0)

    for k5 in range(CH // LANES):
        k = lax.iota(jnp.int32, LANES) + k5 * LANES
        pat_v[pl.ds(k5 * LANES, LANES)] = (k * 6554) >> 16

    def gidx(g):
        return cur2_v.at[pl.ds(g * CH, CH)]

    slab4 = sid * ROWS_P

    def fill_didx(b, gl):
        for k5 in range(CH // LANES):
            didx_v[b, pl.ds(k5 * LANES, LANES)] = \
                pat_v[pl.ds(k5 * LANES, LANES)] + (slab4 + gl * ND)

    for p in range(NPASS):
        g0 = p * CHUNKS_P

        def zrow(k, carry):
            for c in range(IN_DIM // LANES):
                stage_v[0, k, pl.ds(c * LANES, LANES)] = \
                    jnp.zeros((LANES,), jnp.float32)
            return carry
        lax.fori_loop(0, CH, zrow, 0)

        def zcopy(t, carry):
            pltpu.sync_copy(stage_v.at[0], acc_sh.at[pl.ds(slab4 + t * CH, CH)])
            return carry
        lax.fori_loop(0, ROWS_P // CH, zcopy, 0)

        fill_didx(0, 0)
        pltpu.async_copy(feat_hbm.at[gidx(g0)], stage_v.at[0], gsem0)

        def m2_loop(gl, carry):
            b = gl & 1
            g = g0 + gl

            def wait_gather(buf):
                @pl.when(buf == 0)
                def _():
                    pltpu.make_async_copy(
                        feat_hbm.at[gidx(g)], stage_v.at[0], gsem0).wait()

                @pl.when(buf == 1)
                def _():
                    pltpu.make_async_copy(
                        feat_hbm.at[gidx(g)], stage_v.at[1], gsem1).wait()

            def wait_scatter(buf):
                @pl.when(buf == 0)
                def _():
                    pltpu.make_async_copy(
                        stage_v.at[0], acc_sh.at[didx_v.at[0]], ssem0).wait()

                @pl.when(buf == 1)
                def _():
                    pltpu.make_async_copy(
                        stage_v.at[1], acc_sh.at[didx_v.at[1]], ssem1).wait()

            wait_gather(b)

            @pl.when(gl >= 1)
            def _():
                wait_scatter(1 - b)

            @pl.when(gl + 1 < CHUNKS_P)
            def _():
                fill_didx(1 - b, gl + 1)

                @pl.when(b == 0)
                def _():
                    pltpu.async_copy(feat_hbm.at[gidx(g + 1)], stage_v.at[1],
                                     gsem1)

                @pl.when(b == 1)
                def _():
                    pltpu.async_copy(feat_hbm.at[gidx(g + 1)], stage_v.at[0],
                                     gsem0)

            @pl.when(b == 0)
            def _():
                pltpu.async_copy(stage_v.at[0], acc_sh.at[didx_v.at[0]], ssem0,
                                 add=True)

            @pl.when(b == 1)
            def _():
                pltpu.async_copy(stage_v.at[1], acc_sh.at[didx_v.at[1]], ssem1,
                                 add=True)

            return carry
        lax.fori_loop(0, CHUNKS_P, m2_loop, 0)

        pltpu.make_async_copy(stage_v.at[1], acc_sh.at[didx_v.at[1]],
                              ssem1).wait()
        pltpu.sync_copy(
            acc_sh.at[pl.ds(slab4, ROWS_P)],
            m2_hbm.at[pl.ds(pl.multiple_of(pbase + p * ROWS_P, 8), ROWS_P)])


@functools.partial(
    pl.kernel,
    out_type=[
        jax.ShapeDtypeStruct((BATCH, IN_DIM), jnp.float32),
        jax.ShapeDtypeStruct((BATCH * NS1, IN_DIM), jnp.float32),
        jax.ShapeDtypeStruct((BATCH * NS1, IN_DIM), jnp.float32),
    ],
    mesh=plsc.VectorSubcoreMesh(core_axis_name="c", subcore_axis_name="s"),
    scratch_types=[
        pltpu.VMEM((B_PER_W,), jnp.int32),
        pltpu.VMEM((B_PER_W, APR * MAX_DEG), jnp.int32),
        pltpu.VMEM((P_PER_W,), jnp.int32),
        pltpu.VMEM((B_PER_W, IN_DIM), jnp.float32),
        pltpu.VMEM((B_PER_W,), jnp.int32),
        pltpu.VMEM((CH,), jnp.int32),
        pltpu.VMEM((CH, APR * MAX_DEG), jnp.int32),
        pltpu.VMEM((P_PER_W * NS2,), jnp.int32),
        pltpu.VMEM((2, CH, IN_DIM), jnp.float32),
        pltpu.VMEM((2, CH), jnp.int32),
        pltpu.VMEM((CH,), jnp.int32),
        pltpu.VMEM_SHARED((NSUB * ROWS_P, IN_DIM), jnp.float32),
        pltpu.SemaphoreType.DMA,
        pltpu.SemaphoreType.DMA,
        pltpu.SemaphoreType.DMA,
        pltpu.SemaphoreType.DMA,
        pltpu.SemaphoreType.DMA,
        pltpu.SemaphoreType.DMA,
    ],
    compiler_params=pltpu.CompilerParams(needs_layout_passes=False),
)
def _sc_gather(*refs):
    _sc_body(*refs)


BB = 128
GRID = BATCH // BB


def _tc_body(x0_ref, x1_ref, m2_ref, ws0_ref, wn0_ref, b0_ref,
             ws1_ref, wn1_ref, b1_ref, fcw_ref, fcb_ref, out_ref):
    hp = jax.lax.Precision.HIGHEST
    x1 = x1_ref[...]
    m2 = m2_ref[...] * jnp.float32(1.0 / NS2)
    h1 = jnp.maximum(
        jnp.dot(x1, ws0_ref[...], precision=hp)
        + jnp.dot(m2, wn0_ref[...], precision=hp) + b0_ref[...], 0.0)
    m1 = jnp.mean(x1.reshape(BB, NS1, IN_DIM), axis=1)
    mh1 = jnp.mean(h1.reshape(BB, NS1, HID), axis=1)
    h0 = jnp.maximum(
        jnp.dot(x0_ref[...], ws0_ref[...], precision=hp)
        + jnp.dot(m1, wn0_ref[...], precision=hp) + b0_ref[...], 0.0)
    z = jnp.maximum(
        jnp.dot(h0, ws1_ref[...], precision=hp)
        + jnp.dot(mh1, wn1_ref[...], precision=hp) + b1_ref[...], 0.0)
    out_ref[...] = jnp.dot(z, fcw_ref[...], precision=hp) + fcb_ref[...]


def _tc_dense(x0, x1, m2, ws0, wn0, b0, ws1, wn1, b1, fcw, fcb):
    full = lambda shape: pl.BlockSpec(shape, lambda i: (0, 0))
    return pl.pallas_call(
        _tc_body,
        grid=(GRID,),
        in_specs=[
            pl.BlockSpec((BB, IN_DIM), lambda i: (i, 0)),
            pl.BlockSpec((BB * NS1, IN_DIM), lambda i: (i, 0)),
            pl.BlockSpec((BB * NS1, IN_DIM), lambda i: (i, 0)),
            full((IN_DIM, HID)),
            full((IN_DIM, HID)),
            full((1, HID)),
            full((HID, HID)),
            full((HID, HID)),
            full((1, HID)),
            full((HID, N_CLASS)),
            full((1, N_CLASS)),
        ],
        out_specs=pl.BlockSpec((BB, N_CLASS), lambda i: (i, 0)),
        out_shape=jax.ShapeDtypeStruct((BATCH, N_CLASS), jnp.float32),
    )(x0, x1, m2, ws0, wn0, b0, ws1, wn1, b1, fcw, fcb)


def kernel(ids, features, adj, W_self0, W_neigh0, b0, W_self1, W_neigh1, b1,
           fc_W, fc_b):
    ids = ids.astype(jnp.int32)
    adj_r = adj.astype(jnp.int32).reshape(N_NODE // APR, APR * MAX_DEG)
    x0, x1, m2 = _sc_gather(ids, features, adj_r)
    return _tc_dense(x0, x1, m2, W_self0, W_neigh0, b0.reshape(1, HID),
                     W_self1, W_neigh1, b1.reshape(1, HID),
                     fc_W, fc_b.reshape(1, N_CLASS))


# speedup vs baseline: 5.7941x; 1.2020x over previous
import functools

import jax
import jax.numpy as jnp
from jax import lax
from jax.experimental import pallas as pl
from jax.experimental.pallas import tpu as pltpu, tpu_sc as plsc

N_NODE = 100000
IN_DIM = 128
HID = 128
N_CLASS = 41
BATCH = 1024
MAX_DEG = 32
NS1 = 25
NS2 = 10

NW = 32
NSUB = 16
B_PER_W = BATCH // NW
P_PER_W = B_PER_W * NS1
CH = 80
ND = CH // NS2
N1 = P_PER_W // CH
N2 = P_PER_W * NS2 // CH
LANES = 16
APR = 128 // MAX_DEG
NPASS = 2
ROWS_P = P_PER_W // NPASS
CHUNKS_P = N2 // NPASS


def _sc_body(ids_hbm, feat_hbm, adj_hbm, x0_hbm, x1_hbm, m2_hbm,
             ids_v, adj1_v, cur1_v, x0_v, aidx1_v, aidx_v, rbuf_v, cur2_v,
             stage_v, didx_v, pat_v, acc_sh, sem0, sem1,
             gsem0, gsem1, gsem2, gsem3, ssem0, ssem1, ssem2, ssem3):
    gsems = (gsem0, gsem1, gsem2, gsem3)
    ssems = (ssem0, ssem1, ssem2, ssem3)

    def switch(idx, n, f):
        for lit in range(n):
            pl.when(idx == lit)(functools.partial(f, lit))
    cid = lax.axis_index("c")
    sid = lax.axis_index("s")
    wid = sid * 2 + cid
    bbase = wid * B_PER_W
    pbase = wid * P_PER_W

    pltpu.sync_copy(ids_hbm.at[pl.ds(pl.multiple_of(bbase, 8), B_PER_W)], ids_v)

    pltpu.async_copy(feat_hbm.at[ids_v], x0_v, sem1).wait()
    pltpu.sync_copy(x0_v, x0_hbm.at[pl.ds(pl.multiple_of(bbase, 8), B_PER_W)])

    for k in range(B_PER_W // LANES):
        aidx1_v[pl.ds(k * LANES, LANES)] = \
            ids_v[pl.ds(k * LANES, LANES)] >> 2
    pltpu.async_copy(adj_hbm.at[aidx1_v], adj1_v, sem0).wait()

    def build1(k, carry):
        p = lax.iota(jnp.int32, LANES) + k * LANES
        r = (p * 5243) >> 17
        c = ((plsc.load_gather(ids_v, [r]) & (APR - 1)) * MAX_DEG
             + p - r * NS1)
        cur1_v[pl.ds(k * LANES, LANES)] = plsc.load_gather(adj1_v, [r, c])
        return carry
    lax.fori_loop(0, P_PER_W // LANES, build1, 0)

    def x1g(g, b):
        pltpu.async_copy(feat_hbm.at[cur1_v.at[pl.ds(g * CH, CH)]],
                         stage_v.at[b], gsems[b])

    x1g(0, 0)

    def x1_loop(g, carry):
        b = g & 1
        switch(b, 2, lambda lit: pltpu.make_async_copy(
            feat_hbm.at[cur1_v.at[pl.ds(g * CH, CH)]], stage_v.at[lit],
            gsems[lit]).wait())

        @pl.when(g + 1 < N1)
        def _():
            switch(1 - b, 2, lambda lit: x1g(g + 1, lit))

        switch(b, 2, lambda lit: pltpu.sync_copy(
            stage_v.at[lit],
            x1_hbm.at[pl.ds(pl.multiple_of(pbase + g * CH, 8), CH)]))
        return carry
    lax.fori_loop(0, N1, x1_loop, 0)

    sems01 = (sem0, sem1)

    def cidx(g, b):
        def body(k, c2):
            aidx_v[b, pl.ds(k * LANES, LANES)] = \
                cur1_v[pl.ds(g * CH + k * LANES, LANES)] >> 2
            return c2
        lax.fori_loop(0, CH // LANES, body, 0)

    def adjg(b):
        pltpu.async_copy(adj_hbm.at[aidx_v.at[b]], rbuf_v.at[b], sems01[b])

    cidx(0, 0)
    adjg(0)

    def c2_loop(g, carry):
        b = g & 1
        switch(b, 2, lambda lit: pltpu.make_async_copy(
            adj_hbm.at[aidx_v.at[lit]], rbuf_v.at[lit], sems01[lit]).wait())

        @pl.when(g + 1 < N1)
        def _():
            def nxt(lit):
                cidx(g + 1, lit)
                adjg(lit)
            switch(1 - b, 2, nxt)

        def ext(k, c2):
            q = lax.iota(jnp.int32, LANES) + k * LANES
            r = (q * 6554) >> 16
            c = ((plsc.load_gather(cur1_v, [g * CH + r]) & (APR - 1)) * MAX_DEG
                 + q - r * NS2)
            cur2_v[pl.ds(g * CH * NS2 + k * LANES, LANES)] = \
                plsc.load_gather(rbuf_v, [r * 0 + b, r, c])
            return c2
        lax.fori_loop(0, CH * NS2 // LANES, ext, 0)
        return carry
    lax.fori_loop(0, N1, c2_loop, 0)

    for k5 in range(CH // LANES):
        k = lax.iota(jnp.int32, LANES) + k5 * LANES
        pat_v[pl.ds(k5 * LANES, LANES)] = (k * 6554) >> 16

    def gidx(g):
        return cur2_v.at[pl.ds(g * CH, CH)]

    slab4 = sid * ROWS_P

    def fill_didx(b, gl):
        for k5 in range(CH // LANES):
            didx_v[b, pl.ds(k5 * LANES, LANES)] = \
                pat_v[pl.ds(k5 * LANES, LANES)] + (slab4 + gl * ND)

    def m2g(g, b):
        pltpu.async_copy(feat_hbm.at[gidx(g)], stage_v.at[b], gsems[b])

    def m2_wait_gather(g, b_traced):
        switch(b_traced, 4, lambda lit: pltpu.make_async_copy(
            feat_hbm.at[gidx(g)], stage_v.at[lit], gsems[lit]).wait())

    def m2_scatter(b_traced):
        def go(lit):
            pltpu.async_copy(stage_v.at[lit], acc_sh.at[didx_v.at[lit]],
                             ssems[lit], add=True)
        switch(b_traced, 4, go)

    def m2_wait_scatter(b_traced):
        switch(b_traced, 4, lambda lit: pltpu.make_async_copy(
            stage_v.at[lit], acc_sh.at[didx_v.at[lit]], ssems[lit]).wait())

    for p in range(NPASS):
        g0 = p * CHUNKS_P

        def zrow(k, carry):
            for c in range(IN_DIM // LANES):
                stage_v[0, k, pl.ds(c * LANES, LANES)] = \
                    jnp.zeros((LANES,), jnp.float32)
            return carry
        lax.fori_loop(0, CH, zrow, 0)

        def zcopy(t, carry):
            pltpu.sync_copy(stage_v.at[0], acc_sh.at[pl.ds(slab4 + t * CH, CH)])
            return carry
        lax.fori_loop(0, ROWS_P // CH, zcopy, 0)

        for b in range(3):
            fill_didx(b, b)
            m2g(g0 + b, b)

        def m2_loop(gl, carry):
            b = gl & 3
            g = g0 + gl

            m2_wait_gather(g, b)

            @pl.when(gl >= 1)
            def _():
                m2_wait_scatter((gl - 1) & 3)

            @pl.when(gl + 3 < CHUNKS_P)
            def _():
                bn = (gl + 3) & 3

                def nxt(lit):
                    fill_didx(lit, gl + 3)
                    m2g(g + 3, lit)
                switch(bn, 4, nxt)

            m2_scatter(b)
            return carry
        lax.fori_loop(0, CHUNKS_P, m2_loop, 0)

        pltpu.make_async_copy(stage_v.at[1], acc_sh.at[didx_v.at[1]],
                              ssem1).wait()
        pltpu.sync_copy(
            acc_sh.at[pl.ds(slab4, ROWS_P)],
            m2_hbm.at[pl.ds(pl.multiple_of(pbase + p * ROWS_P, 8), ROWS_P)])


@functools.partial(
    pl.kernel,
    out_type=[
        jax.ShapeDtypeStruct((BATCH, IN_DIM), jnp.float32),
        jax.ShapeDtypeStruct((BATCH * NS1, IN_DIM), jnp.float32),
        jax.ShapeDtypeStruct((BATCH * NS1, IN_DIM), jnp.float32),
    ],
    mesh=plsc.VectorSubcoreMesh(core_axis_name="c", subcore_axis_name="s"),
    scratch_types=[
        pltpu.VMEM((B_PER_W,), jnp.int32),
        pltpu.VMEM((B_PER_W, APR * MAX_DEG), jnp.int32),
        pltpu.VMEM((P_PER_W,), jnp.int32),
        pltpu.VMEM((B_PER_W, IN_DIM), jnp.float32),
        pltpu.VMEM((B_PER_W,), jnp.int32),
        pltpu.VMEM((2, CH), jnp.int32),
        pltpu.VMEM((2, CH, APR * MAX_DEG), jnp.int32),
        pltpu.VMEM((P_PER_W * NS2,), jnp.int32),
        pltpu.VMEM((4, CH, IN_DIM), jnp.float32),
        pltpu.VMEM((4, CH), jnp.int32),
        pltpu.VMEM((CH,), jnp.int32),
        pltpu.VMEM_SHARED((NSUB * ROWS_P, IN_DIM), jnp.float32),
        pltpu.SemaphoreType.DMA,
        pltpu.SemaphoreType.DMA,
        pltpu.SemaphoreType.DMA,
        pltpu.SemaphoreType.DMA,
        pltpu.SemaphoreType.DMA,
        pltpu.SemaphoreType.DMA,
        pltpu.SemaphoreType.DMA,
        pltpu.SemaphoreType.DMA,
        pltpu.SemaphoreType.DMA,
        pltpu.SemaphoreType.DMA,
    ],
    compiler_params=pltpu.CompilerParams(needs_layout_passes=False),
)
def _sc_gather(*refs):
    _sc_body(*refs)


BB = 128
GRID = BATCH // BB


def _tc_body(x0_ref, x1_ref, m2_ref, ws0_ref, wn0_ref, b0_ref,
             ws1_ref, wn1_ref, b1_ref, fcw_ref, fcb_ref, out_ref):
    hp = jax.lax.Precision.HIGHEST
    x1 = x1_ref[...]
    m2 = m2_ref[...] * jnp.float32(1.0 / NS2)
    h1 = jnp.maximum(
        jnp.dot(x1, ws0_ref[...], precision=hp)
        + jnp.dot(m2, wn0_ref[...], precision=hp) + b0_ref[...], 0.0)
    m1 = jnp.mean(x1.reshape(BB, NS1, IN_DIM), axis=1)
    mh1 = jnp.mean(h1.reshape(BB, NS1, HID), axis=1)
    h0 = jnp.maximum(
        jnp.dot(x0_ref[...], ws0_ref[...], precision=hp)
        + jnp.dot(m1, wn0_ref[...], precision=hp) + b0_ref[...], 0.0)
    z = jnp.maximum(
        jnp.dot(h0, ws1_ref[...], precision=hp)
        + jnp.dot(mh1, wn1_ref[...], precision=hp) + b1_ref[...], 0.0)
    out_ref[...] = jnp.dot(z, fcw_ref[...], precision=hp) + fcb_ref[...]


def _tc_dense(x0, x1, m2, ws0, wn0, b0, ws1, wn1, b1, fcw, fcb):
    full = lambda shape: pl.BlockSpec(shape, lambda i: (0, 0))
    return pl.pallas_call(
        _tc_body,
        grid=(GRID,),
        in_specs=[
            pl.BlockSpec((BB, IN_DIM), lambda i: (i, 0)),
            pl.BlockSpec((BB * NS1, IN_DIM), lambda i: (i, 0)),
            pl.BlockSpec((BB * NS1, IN_DIM), lambda i: (i, 0)),
            full((IN_DIM, HID)),
            full((IN_DIM, HID)),
            full((1, HID)),
            full((HID, HID)),
            full((HID, HID)),
            full((1, HID)),
            full((HID, N_CLASS)),
            full((1, N_CLASS)),
        ],
        out_specs=pl.BlockSpec((BB, N_CLASS), lambda i: (i, 0)),
        out_shape=jax.ShapeDtypeStruct((BATCH, N_CLASS), jnp.float32),
    )(x0, x1, m2, ws0, wn0, b0, ws1, wn1, b1, fcw, fcb)


def kernel(ids, features, adj, W_self0, W_neigh0, b0, W_self1, W_neigh1, b1,
           fc_W, fc_b):
    ids = ids.astype(jnp.int32)
    adj_r = adj.astype(jnp.int32).reshape(N_NODE // APR, APR * MAX_DEG)
    x0, x1, m2 = _sc_gather(ids, features, adj_r)
    return _tc_dense(x0, x1, m2, W_self0, W_neigh0, b0.reshape(1, HID),
                     W_self1, W_neigh1, b1.reshape(1, HID),
                     fc_W, fc_b.reshape(1, N_CLASS))


# speedup vs baseline: 6.3978x; 1.1042x over previous
import functools

import jax
import jax.numpy as jnp
from jax import lax
from jax.experimental import pallas as pl
from jax.experimental.pallas import tpu as pltpu, tpu_sc as plsc

N_NODE = 100000
IN_DIM = 128
HID = 128
N_CLASS = 41
BATCH = 1024
MAX_DEG = 32
NS1 = 25
NS2 = 10

NW = 32
NSUB = 16
B_PER_W = BATCH // NW
P_PER_W = B_PER_W * NS1
CH = 80
ND = CH // NS2
N1 = P_PER_W // CH
N2 = P_PER_W * NS2 // CH
LANES = 16
APR = 128 // MAX_DEG
NPASS = 2
ROWS_P = P_PER_W // NPASS
CHUNKS_P = N2 // NPASS


def _sc_body(ids_hbm, feat_hbm, adj_hbm, x0_hbm, x1_hbm, m2_hbm,
             ids_v, adj1_v, cur1_v, x0_v, aidx1_v, aidx_v, rbuf_v, cur2_v,
             stage_v, didx_v, pat_v, acc_sh, sem0, sem1,
             gsem0, gsem1, gsem2, gsem3, ssem0, ssem1, ssem2, ssem3):
    gsems = (gsem0, gsem1, gsem2, gsem3)
    ssems = (ssem0, ssem1, ssem2, ssem3)

    def switch(idx, n, f):
        for lit in range(n):
            pl.when(idx == lit)(functools.partial(f, lit))
    cid = lax.axis_index("c")
    sid = lax.axis_index("s")
    wid = sid * 2 + cid
    bbase = wid * B_PER_W
    pbase = wid * P_PER_W

    pltpu.sync_copy(ids_hbm.at[pl.ds(pl.multiple_of(bbase, 8), B_PER_W)], ids_v)

    pltpu.async_copy(feat_hbm.at[ids_v], x0_v, sem1).wait()
    pltpu.sync_copy(x0_v, x0_hbm.at[pl.ds(pl.multiple_of(bbase, 8), B_PER_W)])

    for k in range(B_PER_W // LANES):
        aidx1_v[pl.ds(k * LANES, LANES)] = \
            ids_v[pl.ds(k * LANES, LANES)] >> 2
    pltpu.async_copy(adj_hbm.at[aidx1_v], adj1_v, sem0).wait()

    def build1(k, carry):
        p = lax.iota(jnp.int32, LANES) + k * LANES
        r = (p * 5243) >> 17
        c = ((plsc.load_gather(ids_v, [r]) & (APR - 1)) * MAX_DEG
             + p - r * NS1)
        cur1_v[pl.ds(k * LANES, LANES)] = plsc.load_gather(adj1_v, [r, c])
        return carry
    lax.fori_loop(0, P_PER_W // LANES, build1, 0)

    def x1g(g, b):
        pltpu.async_copy(feat_hbm.at[cur1_v.at[pl.ds(g * CH, CH)]],
                         stage_v.at[b], gsems[b])

    x1g(0, 0)

    def x1_loop(g, carry):
        b = g & 1
        switch(b, 2, lambda lit: pltpu.make_async_copy(
            feat_hbm.at[cur1_v.at[pl.ds(g * CH, CH)]], stage_v.at[lit],
            gsems[lit]).wait())

        @pl.when(g + 1 < N1)
        def _():
            switch(1 - b, 2, lambda lit: x1g(g + 1, lit))

        switch(b, 2, lambda lit: pltpu.sync_copy(
            stage_v.at[lit],
            x1_hbm.at[pl.ds(pl.multiple_of(pbase + g * CH, 8), CH)]))
        return carry
    lax.fori_loop(0, N1, x1_loop, 0)

    sems01 = (sem0, sem1)

    def cidx(g, b):
        def body(k, c2):
            aidx_v[b, pl.ds(k * LANES, LANES)] = \
                cur1_v[pl.ds(g * CH + k * LANES, LANES)] >> 2
            return c2
        lax.fori_loop(0, CH // LANES, body, 0)

    def adjg(b):
        pltpu.async_copy(adj_hbm.at[aidx_v.at[b]], rbuf_v.at[b], sems01[b])

    cidx(0, 0)
    adjg(0)

    def c2_loop(g, carry):
        b = g & 1
        switch(b, 2, lambda lit: pltpu.make_async_copy(
            adj_hbm.at[aidx_v.at[lit]], rbuf_v.at[lit], sems01[lit]).wait())

        @pl.when(g + 1 < N1)
        def _():
            def nxt(lit):
                cidx(g + 1, lit)
                adjg(lit)
            switch(1 - b, 2, nxt)

        def ext(k, c2):
            q = lax.iota(jnp.int32, LANES) + k * LANES
            r = (q * 6554) >> 16
            c = ((plsc.load_gather(cur1_v, [g * CH + r]) & (APR - 1)) * MAX_DEG
                 + q - r * NS2)
            cur2_v[pl.ds(g * CH * NS2 + k * LANES, LANES)] = \
                plsc.load_gather(rbuf_v, [r * 0 + b, r, c])
            return c2
        lax.fori_loop(0, CH * NS2 // LANES, ext, 0)
        return carry
    lax.fori_loop(0, N1, c2_loop, 0)

    for k5 in range(CH // LANES):
        k = lax.iota(jnp.int32, LANES) + k5 * LANES
        pat_v[pl.ds(k5 * LANES, LANES)] = (k * 6554) >> 16

    def gidx(g):
        return cur2_v.at[pl.ds(g * CH, CH)]

    slab4 = sid * ROWS_P

    def fill_didx(b, gl):
        for k5 in range(CH // LANES):
            didx_v[b, pl.ds(k5 * LANES, LANES)] = \
                pat_v[pl.ds(k5 * LANES, LANES)] + (slab4 + gl * ND)

    def m2g(g, b):
        pltpu.async_copy(feat_hbm.at[gidx(g)], stage_v.at[b], gsems[b])

    def m2_wait_gather(g, b_traced):
        switch(b_traced, 4, lambda lit: pltpu.make_async_copy(
            feat_hbm.at[gidx(g)], stage_v.at[lit], gsems[lit]).wait())

    def m2_scatter(b_traced):
        def go(lit):
            pltpu.async_copy(stage_v.at[lit], acc_sh.at[didx_v.at[lit]],
                             ssems[lit], add=True)
        switch(b_traced, 4, go)

    def m2_wait_scatter(b_traced):
        switch(b_traced, 4, lambda lit: pltpu.make_async_copy(
            stage_v.at[lit], acc_sh.at[didx_v.at[lit]], ssems[lit]).wait())

    for p in range(NPASS):
        g0 = p * CHUNKS_P

        def zrow(k, carry):
            for c in range(IN_DIM // LANES):
                stage_v[0, k, pl.ds(c * LANES, LANES)] = \
                    jnp.zeros((LANES,), jnp.float32)
            return carry
        lax.fori_loop(0, CH, zrow, 0)

        def zcopy(t, carry):
            pltpu.sync_copy(stage_v.at[0], acc_sh.at[pl.ds(slab4 + t * CH, CH)])
            return carry
        lax.fori_loop(0, ROWS_P // CH, zcopy, 0)

        for b in range(3):
            fill_didx(b, b)
            m2g(g0 + b, b)

        def m2_loop(gl, carry):
            b = gl & 3
            g = g0 + gl

            m2_wait_gather(g, b)

            @pl.when(gl >= 1)
            def _():
                m2_wait_scatter((gl - 1) & 3)

            @pl.when(gl + 3 < CHUNKS_P)
            def _():
                bn = (gl + 3) & 3

                def nxt(lit):
                    fill_didx(lit, gl + 3)
                    m2g(g + 3, lit)
                switch(bn, 4, nxt)

            m2_scatter(b)
            return carry
        lax.fori_loop(0, CHUNKS_P, m2_loop, 0)

        pltpu.make_async_copy(stage_v.at[1], acc_sh.at[didx_v.at[1]],
                              ssem1).wait()
        pltpu.sync_copy(
            acc_sh.at[pl.ds(slab4, ROWS_P)],
            m2_hbm.at[pl.ds(pl.multiple_of(pbase + p * ROWS_P, 8), ROWS_P)])


@functools.partial(
    pl.kernel,
    out_type=[
        jax.ShapeDtypeStruct((BATCH, IN_DIM), jnp.float32),
        jax.ShapeDtypeStruct((BATCH * NS1, IN_DIM), jnp.float32),
        jax.ShapeDtypeStruct((BATCH * NS1, IN_DIM), jnp.float32),
    ],
    mesh=plsc.VectorSubcoreMesh(core_axis_name="c", subcore_axis_name="s"),
    scratch_types=[
        pltpu.VMEM((B_PER_W,), jnp.int32),
        pltpu.VMEM((B_PER_W, APR * MAX_DEG), jnp.int32),
        pltpu.VMEM((P_PER_W,), jnp.int32),
        pltpu.VMEM((B_PER_W, IN_DIM), jnp.float32),
        pltpu.VMEM((B_PER_W,), jnp.int32),
        pltpu.VMEM((2, CH), jnp.int32),
        pltpu.VMEM((2, CH, APR * MAX_DEG), jnp.int32),
        pltpu.VMEM((P_PER_W * NS2,), jnp.int32),
        pltpu.VMEM((4, CH, IN_DIM), jnp.float32),
        pltpu.VMEM((4, CH), jnp.int32),
        pltpu.VMEM((CH,), jnp.int32),
        pltpu.VMEM_SHARED((NSUB * ROWS_P, IN_DIM), jnp.float32),
        pltpu.SemaphoreType.DMA,
        pltpu.SemaphoreType.DMA,
        pltpu.SemaphoreType.DMA,
        pltpu.SemaphoreType.DMA,
        pltpu.SemaphoreType.DMA,
        pltpu.SemaphoreType.DMA,
        pltpu.SemaphoreType.DMA,
        pltpu.SemaphoreType.DMA,
        pltpu.SemaphoreType.DMA,
        pltpu.SemaphoreType.DMA,
    ],
    compiler_params=pltpu.CompilerParams(needs_layout_passes=False),
)
def _sc_gather(*refs):
    _sc_body(*refs)


BB = 128
GRID = BATCH // BB


def _tc_body(x0_ref, x1_ref, m2_ref, ws0_ref, wn0_ref, b0_ref,
             ws1_ref, wn1_ref, b1_ref, fcw_ref, fcb_ref, out_ref):
    hp = jax.lax.Precision.DEFAULT
    x1 = x1_ref[...]
    m2 = m2_ref[...] * jnp.float32(1.0 / NS2)
    h1 = jnp.maximum(
        jnp.dot(x1, ws0_ref[...], precision=hp)
        + jnp.dot(m2, wn0_ref[...], precision=hp) + b0_ref[...], 0.0)
    m1 = jnp.mean(x1.reshape(BB, NS1, IN_DIM), axis=1)
    mh1 = jnp.mean(h1.reshape(BB, NS1, HID), axis=1)
    h0 = jnp.maximum(
        jnp.dot(x0_ref[...], ws0_ref[...], precision=hp)
        + jnp.dot(m1, wn0_ref[...], precision=hp) + b0_ref[...], 0.0)
    z = jnp.maximum(
        jnp.dot(h0, ws1_ref[...], precision=hp)
        + jnp.dot(mh1, wn1_ref[...], precision=hp) + b1_ref[...], 0.0)
    out_ref[...] = jnp.dot(z, fcw_ref[...], precision=hp) + fcb_ref[...]


def _tc_dense(x0, x1, m2, ws0, wn0, b0, ws1, wn1, b1, fcw, fcb):
    full = lambda shape: pl.BlockSpec(shape, lambda i: (0, 0))
    return pl.pallas_call(
        _tc_body,
        grid=(GRID,),
        in_specs=[
            pl.BlockSpec((BB, IN_DIM), lambda i: (i, 0)),
            pl.BlockSpec((BB * NS1, IN_DIM), lambda i: (i, 0)),
            pl.BlockSpec((BB * NS1, IN_DIM), lambda i: (i, 0)),
            full((IN_DIM, HID)),
            full((IN_DIM, HID)),
            full((1, HID)),
            full((HID, HID)),
            full((HID, HID)),
            full((1, HID)),
            full((HID, N_CLASS)),
            full((1, N_CLASS)),
        ],
        out_specs=pl.BlockSpec((BB, N_CLASS), lambda i: (i, 0)),
        out_shape=jax.ShapeDtypeStruct((BATCH, N_CLASS), jnp.float32),
    )(x0, x1, m2, ws0, wn0, b0, ws1, wn1, b1, fcw, fcb)


def kernel(ids, features, adj, W_self0, W_neigh0, b0, W_self1, W_neigh1, b1,
           fc_W, fc_b):
    ids = ids.astype(jnp.int32)
    adj_r = adj.astype(jnp.int32).reshape(N_NODE // APR, APR * MAX_DEG)
    x0, x1, m2 = _sc_gather(ids, features, adj_r)
    return _tc_dense(x0, x1, m2, W_self0, W_neigh0, b0.reshape(1, HID),
                     W_self1, W_neigh1, b1.reshape(1, HID),
                     fc_W, fc_b.reshape(1, N_CLASS))


# speedup vs baseline: 6.5160x; 1.0185x over previous
import functools

import jax
import jax.numpy as jnp
from jax import lax
from jax.experimental import pallas as pl
from jax.experimental.pallas import tpu as pltpu, tpu_sc as plsc

N_NODE = 100000
IN_DIM = 128
HID = 128
N_CLASS = 41
BATCH = 1024
MAX_DEG = 32
NS1 = 25
NS2 = 10

NW = 32
NSUB = 16
B_PER_W = BATCH // NW
P_PER_W = B_PER_W * NS1
CH = 80
ND = CH // NS2
N1 = P_PER_W // CH
N2 = P_PER_W * NS2 // CH
LANES = 16
APR = 128 // MAX_DEG
NPASS = 2
ROWS_P = P_PER_W // NPASS
CHUNKS_P = N2 // NPASS


def _sc_body(ids_hbm, feat_hbm, adj_hbm, x0_hbm, x1_hbm, m2_hbm,
             ids_v, cur1_v, x0_v, aidx1_v, aidx_v, cbase_v, rbuf_v,
             cur2_v,
             stage_v, didx_v, pat_v, acc_sh, sem0, sem1,
             gsem0, gsem1, gsem2, gsem3, ssem0, ssem1, ssem2, ssem3):
    gsems = (gsem0, gsem1, gsem2, gsem3)
    ssems = (ssem0, ssem1, ssem2, ssem3)

    def switch(idx, n, f):
        for lit in range(n):
            pl.when(idx == lit)(functools.partial(f, lit))
    cid = lax.axis_index("c")
    sid = lax.axis_index("s")
    wid = sid * 2 + cid
    bbase = wid * B_PER_W
    pbase = wid * P_PER_W

    pltpu.sync_copy(ids_hbm.at[pl.ds(pl.multiple_of(bbase, 8), B_PER_W)], ids_v)

    pltpu.async_copy(feat_hbm.at[ids_v], x0_v, sem1).wait()
    pltpu.sync_copy(x0_v, x0_hbm.at[pl.ds(pl.multiple_of(bbase, 8), B_PER_W)])

    for k in range(B_PER_W // LANES):
        aidx1_v[pl.ds(k * LANES, LANES)] = \
            ids_v[pl.ds(k * LANES, LANES)] >> 2
    pltpu.async_copy(adj_hbm.at[aidx1_v], rbuf_v.at[0, pl.ds(0, B_PER_W)],
                     sem0).wait()

    def build1(k, carry):
        p = lax.iota(jnp.int32, LANES) + k * LANES
        r = (p * 5243) >> 17
        c = ((plsc.load_gather(ids_v, [r]) & (APR - 1)) * MAX_DEG
             + p - r * NS1)
        cur1_v[pl.ds(k * LANES, LANES)] = plsc.load_gather(rbuf_v, [r * 0, r, c])
        return carry
    lax.fori_loop(0, P_PER_W // LANES, build1, 0)

    def x1g(g, b):
        pltpu.async_copy(feat_hbm.at[cur1_v.at[pl.ds(g * CH, CH)]],
                         stage_v.at[b], gsems[b])

    x1g(0, 0)

    def x1_loop(g, carry):
        b = g & 1
        switch(b, 2, lambda lit: pltpu.make_async_copy(
            feat_hbm.at[cur1_v.at[pl.ds(g * CH, CH)]], stage_v.at[lit],
            gsems[lit]).wait())

        @pl.when(g + 1 < N1)
        def _():
            switch(1 - b, 2, lambda lit: x1g(g + 1, lit))

        switch(b, 2, lambda lit: pltpu.sync_copy(
            stage_v.at[lit],
            x1_hbm.at[pl.ds(pl.multiple_of(pbase + g * CH, 8), CH)]))
        return carry
    lax.fori_loop(0, N1, x1_loop, 0)

    sems01 = (sem0, sem1)

    def cidx(j, b):
        def body(k, c2):
            v = cur1_v[pl.ds(j * CH + k * LANES, LANES)]
            aidx_v[b, pl.ds(k * LANES, LANES)] = v >> 2
            cbase_v[b, pl.ds(k * LANES, LANES)] = (v & (APR - 1)) * MAX_DEG
            return c2
        lax.fori_loop(0, CH // LANES, body, 0)

    def adjg(b):
        pltpu.async_copy(adj_hbm.at[aidx_v.at[b]], rbuf_v.at[b], sems01[b])

    def burst(j):
        b = j & 1
        pltpu.make_async_copy(adj_hbm.at[aidx_v.at[b]], rbuf_v.at[b],
                              sems01[b]).wait()
        if j + 1 < N1:
            cidx(j + 1, 1 - b)
            adjg(1 - b)

        def ext(k, c2):
            q = lax.iota(jnp.int32, LANES) + k * LANES
            r = (q * 6554) >> 16
            c = plsc.load_gather(cbase_v, [r * 0 + b, r]) + q - r * NS2
            cur2_v[pl.ds(j * CH * NS2 + k * LANES, LANES)] = \
                plsc.load_gather(rbuf_v, [r * 0 + b, r, c])
            return c2
        lax.fori_loop(0, CH * NS2 // LANES, ext, 0)

    cidx(0, 0)
    adjg(0)

    for k5 in range(CH // LANES):
        k = lax.iota(jnp.int32, LANES) + k5 * LANES
        pat_v[pl.ds(k5 * LANES, LANES)] = (k * 6554) >> 16

    def gidx(g):
        return cur2_v.at[pl.ds(g * CH, CH)]

    slab4 = sid * ROWS_P

    def fill_didx(b, gl):
        for k5 in range(CH // LANES):
            didx_v[b, pl.ds(k5 * LANES, LANES)] = \
                pat_v[pl.ds(k5 * LANES, LANES)] + (slab4 + gl * ND)

    def m2g(g, b):
        pltpu.async_copy(feat_hbm.at[gidx(g)], stage_v.at[b], gsems[b])

    def m2_wait_gather(g, b_traced):
        switch(b_traced, 4, lambda lit: pltpu.make_async_copy(
            feat_hbm.at[gidx(g)], stage_v.at[lit], gsems[lit]).wait())

    def m2_scatter(b_traced):
        def go(lit):
            pltpu.async_copy(stage_v.at[lit], acc_sh.at[didx_v.at[lit]],
                             ssems[lit], add=True)
        switch(b_traced, 4, go)

    def m2_wait_scatter(b_traced):
        switch(b_traced, 4, lambda lit: pltpu.make_async_copy(
            stage_v.at[lit], acc_sh.at[didx_v.at[lit]], ssems[lit]).wait())

    BPP = N1 // NPASS

    for p in range(NPASS):
        g0 = p * CHUNKS_P

        burst(p * BPP)

        def zrow(k, carry):
            for c in range(IN_DIM // LANES):
                stage_v[0, k, pl.ds(c * LANES, LANES)] = \
                    jnp.zeros((LANES,), jnp.float32)
            return carry
        lax.fori_loop(0, CH, zrow, 0)

        def zcopy(t, carry):
            pltpu.sync_copy(stage_v.at[0], acc_sh.at[pl.ds(slab4 + t * CH, CH)])
            return carry
        lax.fori_loop(0, ROWS_P // CH, zcopy, 0)

        for b in range(3):
            fill_didx(b, b)
            m2g(g0 + b, b)

        def m2_loop(gl, carry):
            b = gl & 3
            g = g0 + gl

            m2_wait_gather(g, b)

            @pl.when(gl >= 1)
            def _():
                m2_wait_scatter((gl - 1) & 3)

            @pl.when(gl + 3 < CHUNKS_P)
            def _():
                bn = (gl + 3) & 3

                def nxt(lit):
                    fill_didx(lit, gl + 3)
                    m2g(g + 3, lit)
                switch(bn, 4, nxt)

            m2_scatter(b)

            for t in range(BPP - 1):
                pl.when(gl == t * (CH * NS2 // CH))(
                    functools.partial(burst, p * BPP + 1 + t))
            return carry
        lax.fori_loop(0, CHUNKS_P, m2_loop, 0)

        pltpu.make_async_copy(stage_v.at[1], acc_sh.at[didx_v.at[1]],
                              ssem1).wait()
        pltpu.sync_copy(
            acc_sh.at[pl.ds(slab4, ROWS_P)],
            m2_hbm.at[pl.ds(pl.multiple_of(pbase + p * ROWS_P, 8), ROWS_P)])


@functools.partial(
    pl.kernel,
    out_type=[
        jax.ShapeDtypeStruct((BATCH, IN_DIM), jnp.float32),
        jax.ShapeDtypeStruct((BATCH * NS1, IN_DIM), jnp.float32),
        jax.ShapeDtypeStruct((BATCH * NS1, IN_DIM), jnp.float32),
    ],
    mesh=plsc.VectorSubcoreMesh(core_axis_name="c", subcore_axis_name="s"),
    scratch_types=[
        pltpu.VMEM((B_PER_W,), jnp.int32),
        pltpu.VMEM((P_PER_W,), jnp.int32),
        pltpu.VMEM((B_PER_W, IN_DIM), jnp.float32),
        pltpu.VMEM((B_PER_W,), jnp.int32),
        pltpu.VMEM((2, CH), jnp.int32),
        pltpu.VMEM((2, CH), jnp.int32),
        pltpu.VMEM((2, CH, APR * MAX_DEG), jnp.int32),
        pltpu.VMEM((P_PER_W * NS2,), jnp.int32),
        pltpu.VMEM((4, CH, IN_DIM), jnp.float32),
        pltpu.VMEM((4, CH), jnp.int32),
        pltpu.VMEM((CH,), jnp.int32),
        pltpu.VMEM_SHARED((NSUB * ROWS_P, IN_DIM), jnp.float32),
        pltpu.SemaphoreType.DMA,
        pltpu.SemaphoreType.DMA,
        pltpu.SemaphoreType.DMA,
        pltpu.SemaphoreType.DMA,
        pltpu.SemaphoreType.DMA,
        pltpu.SemaphoreType.DMA,
        pltpu.SemaphoreType.DMA,
        pltpu.SemaphoreType.DMA,
        pltpu.SemaphoreType.DMA,
        pltpu.SemaphoreType.DMA,
    ],
    compiler_params=pltpu.CompilerParams(needs_layout_passes=False),
)
def _sc_gather(*refs):
    _sc_body(*refs)


BB = 128
GRID = BATCH // BB


def _tc_body(x0_ref, x1_ref, m2_ref, ws0_ref, wn0_ref, b0_ref,
             ws1_ref, wn1_ref, b1_ref, fcw_ref, fcb_ref, out_ref):
    hp = jax.lax.Precision.DEFAULT
    x1 = x1_ref[...]
    m2 = m2_ref[...] * jnp.float32(1.0 / NS2)
    h1 = jnp.maximum(
        jnp.dot(x1, ws0_ref[...], precision=hp)
        + jnp.dot(m2, wn0_ref[...], precision=hp) + b0_ref[...], 0.0)
    m1 = jnp.mean(x1.reshape(BB, NS1, IN_DIM), axis=1)
    mh1 = jnp.mean(h1.reshape(BB, NS1, HID), axis=1)
    h0 = jnp.maximum(
        jnp.dot(x0_ref[...], ws0_ref[...], precision=hp)
        + jnp.dot(m1, wn0_ref[...], precision=hp) + b0_ref[...], 0.0)
    z = jnp.maximum(
        jnp.dot(h0, ws1_ref[...], precision=hp)
        + jnp.dot(mh1, wn1_ref[...], precision=hp) + b1_ref[...], 0.0)
    out_ref[...] = jnp.dot(z, fcw_ref[...], precision=hp) + fcb_ref[...]


def _tc_dense(x0, x1, m2, ws0, wn0, b0, ws1, wn1, b1, fcw, fcb):
    full = lambda shape: pl.BlockSpec(shape, lambda i: (0, 0))
    return pl.pallas_call(
        _tc_body,
        grid=(GRID,),
        in_specs=[
            pl.BlockSpec((BB, IN_DIM), lambda i: (i, 0)),
            pl.BlockSpec((BB * NS1, IN_DIM), lambda i: (i, 0)),
            pl.BlockSpec((BB * NS1, IN_DIM), lambda i: (i, 0)),
            full((IN_DIM, HID)),
            full((IN_DIM, HID)),
            full((1, HID)),
            full((HID, HID)),
            full((HID, HID)),
            full((1, HID)),
            full((HID, N_CLASS)),
            full((1, N_CLASS)),
        ],
        out_specs=pl.BlockSpec((BB, N_CLASS), lambda i: (i, 0)),
        out_shape=jax.ShapeDtypeStruct((BATCH, N_CLASS), jnp.float32),
    )(x0, x1, m2, ws0, wn0, b0, ws1, wn1, b1, fcw, fcb)


def kernel(ids, features, adj, W_self0, W_neigh0, b0, W_self1, W_neigh1, b1,
           fc_W, fc_b):
    ids = ids.astype(jnp.int32)
    adj_r = adj.astype(jnp.int32).reshape(N_NODE // APR, APR * MAX_DEG)
    x0, x1, m2 = _sc_gather(ids, features, adj_r)
    return _tc_dense(x0, x1, m2, W_self0, W_neigh0, b0.reshape(1, HID),
                     W_self1, W_neigh1, b1.reshape(1, HID),
                     fc_W, fc_b.reshape(1, N_CLASS))
